# baseline (speedup 1.0000x reference)
import functools

import jax
import jax.numpy as jnp
from jax import lax
from jax.experimental import pallas as pl
from jax.experimental.pallas import tpu as pltpu
from jax.experimental.pallas import tpu_sc as plsc

N = 10000
D = 128
E = 320000
L = 100000

NC = 2
NS = 16
CHUNK = 128
ACC_ROWS = 10112
SLAB = ACC_ROWS // NS
DUMMY = 10008

NCH_TILE = 160
E_PAD = NS * NCH_TILE * CHUNK
GRP = 8
N_GRP = NCH_TILE // GRP
N_PAIR = N_GRP // 2

L_CH_TILE = 49
L_PER_TILE = L_CH_TILE * CHUNK
L_PAD = NS * L_PER_TILE


def _make_agg_kernel(with_counts):
    mesh = plsc.VectorSubcoreMesh(core_axis_name="c", subcore_axis_name="s")
    out_type = [jax.ShapeDtypeStruct((NC, ACC_ROWS, D), jnp.float32)]
    scratch = [
        pltpu.VMEM((GRP, CHUNK), jnp.int32),
        pltpu.VMEM((GRP, CHUNK), jnp.int32),
        pltpu.VMEM((GRP, CHUNK), jnp.int32),
        pltpu.VMEM((GRP, CHUNK), jnp.int32),
        pltpu.VMEM((CHUNK, D), jnp.float32),
        pltpu.VMEM((CHUNK, D), jnp.float32),
        pltpu.VMEM_SHARED((ACC_ROWS, D), jnp.float32),
        pltpu.SemaphoreType.DMA,
        pltpu.SemaphoreType.DMA,
        pltpu.SemaphoreType.DMA,
        pltpu.SemaphoreType.DMA,
        pltpu.SemaphoreType.DMA,
        pltpu.SemaphoreType.DMA,
    ]
    if with_counts:
        out_type.append(jax.ShapeDtypeStruct((NC, NS, ACC_ROWS), jnp.float32))
        scratch.append(pltpu.VMEM((ACC_ROWS,), jnp.float32))

    def body(*refs):
        if with_counts:
            (table, src_all, dst_all, zf, agg_out, cnt_out,
             src_a, dst_a, src_b, dst_b, buf_a, buf_b, acc,
             sem_da, sem_db, sem_ia, sem_ib, sem_sa, sem_sb, hist) = refs
        else:
            (table, src_all, dst_all, zf, agg_out,
             src_a, dst_a, src_b, dst_b, buf_a, buf_b, acc,
             sem_da, sem_db, sem_ia, sem_ib, sem_sa, sem_sb) = refs
        cid = lax.axis_index("c")
        sid = lax.axis_index("s")
        slab = pl.ds(pl.multiple_of(sid * SLAB, 8), SLAB)
        pltpu.sync_copy(zf.at[slab], acc.at[slab])
        if with_counts:
            zeros16 = jnp.zeros((16,), jnp.float32)

            @pl.loop(0, ACC_ROWS // 16)
            def _(i):
                hist[pl.ds(i * 16, 16)] = zeros16
        plsc.subcore_barrier()

        ch_base = sid * NCH_TILE

        def stage(grp, sb, db, sem):
            off = pl.multiple_of(ch_base + grp * GRP, 8)
            pltpu.async_copy(src_all.at[cid, pl.ds(off, GRP)], sb, sem)
            pltpu.async_copy(dst_all.at[cid, pl.ds(off, GRP)], db, sem)

        def wait_stage(sb, db, sem):
            pltpu.make_async_copy(src_all.at[cid, pl.ds(0, GRP)], sb,
                                  sem).wait()
            pltpu.make_async_copy(src_all.at[cid, pl.ds(0, GRP)], db,
                                  sem).wait()

        def gather(idx_row, buf, sem):
            pltpu.async_copy(table.at[idx_row], buf, sem)

        def wait_data(buf, sem):
            pltpu.make_async_copy(table.at[src_a.at[0]], buf, sem).wait()

        ones16 = jnp.ones((16,), jnp.float32)

        def start_scatter(d_ref, k, buf, ssem):
            pltpu.async_copy(buf, acc.at[d_ref.at[k]], ssem, add=True)
            if with_counts:
                for q in range(CHUNK // 16):
                    idx16 = d_ref[k, pl.ds(q * 16, 16)]
                    plsc.addupdate_scatter(hist, [idx16], ones16)

        def wait_scatter(d_ref, buf, ssem):
            pltpu.make_async_copy(buf, acc.at[d_ref.at[0]], ssem).wait()

        stage(0, src_a, dst_a, sem_ia)
        wait_stage(src_a, dst_a, sem_ia)
        gather(src_a.at[0], buf_a, sem_da)

        @pl.loop(0, N_PAIR)
        def _(gp):
            def guarded(cond, fn):
                if cond is True:
                    fn()
                else:
                    pl.when(cond)(fn)

            def half(s_cur, d_cur, s_nxt, d_nxt, sem_nxt, first_guard,
                     stage_next, nxt_staged):
                for k in range(GRP):
                    if k % 2 == 0:
                        buf, gsem, ssem = buf_a, sem_da, sem_sa
                        obuf, ogsem, ossem = buf_b, sem_db, sem_sb
                    else:
                        buf, gsem, ssem = buf_b, sem_db, sem_sb
                        obuf, ogsem, ossem = buf_a, sem_da, sem_sa
                    wait_data(buf, gsem)
                    start_scatter(d_cur, k, buf, ssem)
                    if k == 0:
                        guarded(first_guard, functools.partial(
                            wait_scatter, d_cur, obuf, ossem))
                        stage_next()
                    else:
                        wait_scatter(d_cur, obuf, ossem)
                    if k < GRP - 1:
                        gather(s_cur.at[k + 1], obuf, ogsem)
                    else:
                        def cross():
                            wait_stage(s_nxt, d_nxt, sem_nxt)
                            gather(s_nxt.at[0], obuf, ogsem)
                        guarded(nxt_staged, cross)

            not_last = gp + 1 < N_PAIR
            half(src_a, dst_a, src_b, dst_b, sem_ib, gp > 0,
                 lambda: stage(2 * gp + 1, src_b, dst_b, sem_ib), True)
            half(src_b, dst_b, src_a, dst_a, sem_ia, True,
                 lambda: guarded(not_last, functools.partial(
                     stage, 2 * gp + 2, src_a, dst_a, sem_ia)),
                 not_last)

        wait_scatter(dst_b, buf_b, sem_sb)
        plsc.subcore_barrier()
        pltpu.sync_copy(acc.at[slab], agg_out.at[cid, slab])
        if with_counts:
            pltpu.sync_copy(hist, cnt_out.at[cid, sid])

    return pl.kernel(body, out_type=out_type, mesh=mesh,
                     scratch_types=scratch,
                     compiler_params=pltpu.CompilerParams(
                         needs_layout_passes=False))


def _make_label_gather_kernel():
    mesh = plsc.VectorSubcoreMesh(core_axis_name="c", subcore_axis_name="s")
    out_type = [jax.ShapeDtypeStruct((NC, L_PAD, D), jnp.float32)]
    scratch = [
        pltpu.VMEM((L_PER_TILE,), jnp.int32),
        pltpu.VMEM((CHUNK, D), jnp.float32),
        pltpu.VMEM((CHUNK, D), jnp.float32),
        pltpu.SemaphoreType.DMA,
        pltpu.SemaphoreType.DMA,
    ]

    def body(table, idx_all, out, idx_v, buf_a, buf_b, sem_a, sem_b):
        cid = lax.axis_index("c")
        sid = lax.axis_index("s")
        base = pl.multiple_of(sid * L_PER_TILE, CHUNK)
        pltpu.sync_copy(idx_all.at[cid, pl.ds(base, L_PER_TILE)], idx_v)

        def start(j, buf, sem):
            pltpu.async_copy(table.at[idx_v.at[pl.ds(j * CHUNK, CHUNK)]],
                             buf, sem)

        def wait(buf, sem):
            pltpu.make_async_copy(table.at[idx_v.at[pl.ds(0, CHUNK)]],
                                  buf, sem).wait()

        def put(j, buf):
            pltpu.sync_copy(buf, out.at[cid, pl.ds(base + j * CHUNK, CHUNK)])

        start(0, buf_a, sem_a)
        start(1, buf_b, sem_b)

        @pl.loop(0, (L_CH_TILE - 1) // 2)
        def _(g):
            j0 = g * 2
            j1 = j0 + 1
            wait(buf_a, sem_a)
            put(j0, buf_a)

            @pl.when(j0 + 2 < L_CH_TILE)
            def _():
                start(j0 + 2, buf_a, sem_a)

            wait(buf_b, sem_b)
            put(j1, buf_b)

            @pl.when(j1 + 2 < L_CH_TILE)
            def _():
                start(j1 + 2, buf_b, sem_b)

        wait(buf_a, sem_a)
        put(L_CH_TILE - 1, buf_a)

    return pl.kernel(body, out_type=out_type, mesh=mesh,
                     scratch_types=scratch)


_agg_with_counts = _make_agg_kernel(True)
_agg_plain = _make_agg_kernel(False)
_label_gather = _make_label_gather_kernel()


def _sage_update_body(relu, agg, cnt, x, wl, wr, b, o):
    count = jnp.maximum(jnp.sum(cnt[0], axis=0), 1.0)[:, None]
    mean = agg[0] / count
    acc = (jnp.dot(mean, wl[0], preferred_element_type=jnp.float32) + b[0]
           + jnp.dot(x[0], wr[0], preferred_element_type=jnp.float32))
    o[0] = jnp.maximum(acc, 0.0) if relu else acc


def _sage_update(agg, cnt, x, wl, wr, b, relu):
    return pl.pallas_call(
        functools.partial(_sage_update_body, relu),
        grid=(2,),
        in_specs=[
            pl.BlockSpec((1, ACC_ROWS, D), lambda t: (t, 0, 0)),
            pl.BlockSpec((1, NS, ACC_ROWS), lambda t: (t, 0, 0)),
            pl.BlockSpec((1, ACC_ROWS, D), lambda t: (t, 0, 0)),
            pl.BlockSpec((1, D, D), lambda t: (t, 0, 0)),
            pl.BlockSpec((1, D, D), lambda t: (t, 0, 0)),
            pl.BlockSpec((1, 1, D), lambda t: (t, 0, 0)),
        ],
        out_specs=pl.BlockSpec((1, ACC_ROWS, D), lambda t: (t, 0, 0)),
        out_shape=jax.ShapeDtypeStruct((2, ACC_ROWS, D), jnp.float32),
    )(agg, cnt, x, wl, wr, b)


def _decoder_body(za, zh, w1a, w1b, b1, w2, b2, z1o, z2o, z3o):
    a = za[0]
    h = zh[0]
    z2 = jnp.maximum(
        jnp.dot(a, w1a[...], preferred_element_type=jnp.float32)
        + jnp.dot(h, w1b[...], preferred_element_type=jnp.float32)
        + b1[...], 0.0)
    z1o[:, 0:D] = a
    z1o[:, D:2 * D] = h
    z2o[...] = z2
    z3o[...] = jnp.sum(z2 * w2[...], axis=1, keepdims=True) + b2[...]


def _decoder(zparts, w1a, w1b, b1, w2row, b2, rows_block=2000):
    nb = L // rows_block
    return pl.pallas_call(
        _decoder_body,
        grid=(nb,),
        in_specs=[
            pl.BlockSpec((1, rows_block, D), lambda i: (0, i, 0)),
            pl.BlockSpec((1, rows_block, D), lambda i: (1, i, 0)),
            pl.BlockSpec((D, D), lambda i: (0, 0)),
            pl.BlockSpec((D, D), lambda i: (0, 0)),
            pl.BlockSpec((1, D), lambda i: (0, 0)),
            pl.BlockSpec((1, D), lambda i: (0, 0)),
            pl.BlockSpec((1, 1), lambda i: (0, 0)),
        ],
        out_specs=[
            pl.BlockSpec((rows_block, 2 * D), lambda i: (i, 0)),
            pl.BlockSpec((rows_block, D), lambda i: (i, 0)),
            pl.BlockSpec((rows_block, 1), lambda i: (i, 0)),
        ],
        out_shape=[
            jax.ShapeDtypeStruct((L, 2 * D), jnp.float32),
            jax.ShapeDtypeStruct((L, D), jnp.float32),
            jax.ShapeDtypeStruct((L, 1), jnp.float32),
        ],
    )(zparts, zparts, w1a, w1b, b1, w2row, b2)


def _pad_idx(idx, pad_value, total):
    pad = total - idx.shape[0]
    return jnp.concatenate(
        [idx.astype(jnp.int32),
         jnp.full((pad,), pad_value, dtype=jnp.int32)])


def kernel(x_author, x_hotel, edge_index_author_hotel,
           edge_index_hotel_author, edge_label_index,
           Wl_ah0, Wr_ah0, b_ah0, Wl_ha0, Wr_ha0, b_ha0,
           Wl_ah1, Wr_ah1, b_ah1, Wl_ha1, Wr_ha1, b_ha1,
           dec_W1, dec_b1, dec_W2, dec_b2):
    x_stack = jnp.concatenate(
        [jnp.stack([x_author, x_hotel]),
         jnp.zeros((2, ACC_ROWS - N, D), jnp.float32)], axis=1)
    table0 = x_stack.reshape(2 * ACC_ROWS, D)

    src0 = _pad_idx(edge_index_hotel_author[0] + ACC_ROWS, 0, E_PAD)
    dst0 = _pad_idx(edge_index_hotel_author[1], DUMMY, E_PAD)
    src1 = _pad_idx(edge_index_author_hotel[0], 0, E_PAD)
    dst1 = _pad_idx(edge_index_author_hotel[1], DUMMY, E_PAD)
    src_all = jnp.stack([src0, src1]).reshape(NC, E_PAD // CHUNK, CHUNK)
    dst_all = jnp.stack([dst0, dst1]).reshape(NC, E_PAD // CHUNK, CHUNK)

    zf = jnp.zeros((ACC_ROWS, D), jnp.float32)

    agg0, cnt = _agg_with_counts(table0, src_all, dst_all, zf)
    wl0 = jnp.stack([Wl_ha0, Wl_ah0])
    wr0 = jnp.stack([Wr_ha0, Wr_ah0])
    bb0 = jnp.stack([b_ha0, b_ah0]).reshape(2, 1, D)
    h_stack = _sage_update(agg0, cnt, x_stack, wl0, wr0, bb0, relu=True)

    (agg1,) = _agg_plain(h_stack.reshape(2 * ACC_ROWS, D), src_all,
                         dst_all, zf)
    wl1 = jnp.stack([Wl_ha1, Wl_ah1])
    wr1 = jnp.stack([Wr_ha1, Wr_ah1])
    bb1 = jnp.stack([b_ha1, b_ah1]).reshape(2, 1, D)
    z_stack = _sage_update(agg1, cnt, h_stack, wl1, wr1, bb1, relu=False)

    row = _pad_idx(edge_label_index[0], 0, L_PAD)
    col = _pad_idx(edge_label_index[1] + ACC_ROWS, 0, L_PAD)
    idx_all = jnp.stack([row, col])
    (zparts,) = _label_gather(z_stack.reshape(2 * ACC_ROWS, D), idx_all)

    z1, z2, z3 = _decoder(zparts, dec_W1[:D], dec_W1[D:],
                          dec_b1.reshape(1, D), dec_W2.reshape(1, D),
                          dec_b2.reshape(1, 1))
    return (z3.reshape(-1), z1, z2)


# speedup vs baseline: 1.0472x; 1.0472x over previous
import functools

import jax
import jax.numpy as jnp
from jax import lax
from jax.experimental import pallas as pl
from jax.experimental.pallas import tpu as pltpu
from jax.experimental.pallas import tpu_sc as plsc

N = 10000
D = 128
E = 320000
L = 100000

NC = 2
NS = 16
CHUNK = 128
ACC_ROWS = 10112
SLAB = ACC_ROWS // NS
DUMMY = 10008

NCH_TILE = 160
E_PAD = NS * NCH_TILE * CHUNK
GRP = 8
N_GRP = NCH_TILE // GRP
N_PAIR = N_GRP // 2

L_CH_TILE = 49
L_PER_TILE = L_CH_TILE * CHUNK
L_PAD = NS * L_PER_TILE


def _make_agg_kernel(with_counts):
    mesh = plsc.VectorSubcoreMesh(core_axis_name="c", subcore_axis_name="s")
    out_type = [jax.ShapeDtypeStruct((NC, ACC_ROWS, D), jnp.float32)]
    scratch = [
        pltpu.VMEM((GRP, CHUNK), jnp.int32),
        pltpu.VMEM((GRP, CHUNK), jnp.int32),
        pltpu.VMEM((GRP, CHUNK), jnp.int32),
        pltpu.VMEM((GRP, CHUNK), jnp.int32),
        pltpu.VMEM((CHUNK, D), jnp.float32),
        pltpu.VMEM((CHUNK, D), jnp.float32),
        pltpu.VMEM_SHARED((ACC_ROWS, D), jnp.float32),
        pltpu.SemaphoreType.DMA,
        pltpu.SemaphoreType.DMA,
        pltpu.SemaphoreType.DMA,
        pltpu.SemaphoreType.DMA,
    ]
    if with_counts:
        out_type.append(jax.ShapeDtypeStruct((NC, NS, ACC_ROWS), jnp.float32))
        scratch.append(pltpu.VMEM((ACC_ROWS,), jnp.float32))

    def body(*refs):
        if with_counts:
            (table, src_all, dst_all, zf, agg_out, cnt_out,
             src_a, dst_a, src_b, dst_b, buf_a, buf_b, acc,
             sem_da, sem_db, sem_ia, sem_ib, hist) = refs
        else:
            (table, src_all, dst_all, zf, agg_out,
             src_a, dst_a, src_b, dst_b, buf_a, buf_b, acc,
             sem_da, sem_db, sem_ia, sem_ib) = refs
        cid = lax.axis_index("c")
        sid = lax.axis_index("s")
        slab = pl.ds(pl.multiple_of(sid * SLAB, 8), SLAB)
        pltpu.sync_copy(zf.at[slab], acc.at[slab])
        if with_counts:
            zeros16 = jnp.zeros((16,), jnp.float32)

            @pl.loop(0, ACC_ROWS // 16)
            def _(i):
                hist[pl.ds(i * 16, 16)] = zeros16
        plsc.subcore_barrier()

        ch_base = sid * NCH_TILE

        def stage(grp, sb, db, sem):
            off = pl.multiple_of(ch_base + grp * GRP, 8)
            pltpu.async_copy(src_all.at[cid, pl.ds(off, GRP)], sb, sem)
            pltpu.async_copy(dst_all.at[cid, pl.ds(off, GRP)], db, sem)

        def wait_stage(sb, db, sem):
            pltpu.make_async_copy(src_all.at[cid, pl.ds(0, GRP)], sb,
                                  sem).wait()
            pltpu.make_async_copy(src_all.at[cid, pl.ds(0, GRP)], db,
                                  sem).wait()

        def gather(idx_row, buf, sem):
            pltpu.async_copy(table.at[idx_row], buf, sem)

        def wait_data(buf, sem):
            pltpu.make_async_copy(table.at[src_a.at[0]], buf, sem).wait()

        ones16 = jnp.ones((16,), jnp.float32)

        def scatter(d_ref, k, buf):
            pltpu.sync_copy(buf, acc.at[d_ref.at[k]], add=True)
            if with_counts:
                for q in range(CHUNK // 16):
                    idx16 = d_ref[k, pl.ds(q * 16, 16)]
                    plsc.addupdate_scatter(hist, [idx16], ones16)

        stage(0, src_a, dst_a, sem_ia)
        stage(1, src_b, dst_b, sem_ib)
        wait_stage(src_a, dst_a, sem_ia)
        gather(src_a.at[0], buf_a, sem_da)
        gather(src_a.at[1], buf_b, sem_db)

        @pl.loop(0, N_PAIR)
        def _(gp):
            def guarded(cond, fn):
                if cond is True:
                    fn()
                else:
                    pl.when(cond)(fn)

            def half(s_cur, d_cur, s_nxt, d_nxt, sem_nxt, nxt_staged):
                for k in range(GRP):
                    buf, semd = ((buf_a, sem_da) if k % 2 == 0
                                 else (buf_b, sem_db))
                    wait_data(buf, semd)
                    scatter(d_cur, k, buf)
                    if k == GRP - 2:
                        guarded(nxt_staged,
                                lambda: wait_stage(s_nxt, d_nxt, sem_nxt))
                    if k < GRP - 2:
                        gather(s_cur.at[k + 2], buf, semd)
                    else:
                        guarded(nxt_staged, functools.partial(
                            gather, s_nxt.at[k + 2 - GRP], buf, semd))

            not_last = gp + 1 < N_PAIR
            half(src_a, dst_a, src_b, dst_b, sem_ib, True)

            @pl.when(not_last)
            def _():
                stage(2 * gp + 2, src_a, dst_a, sem_ia)

            half(src_b, dst_b, src_a, dst_a, sem_ia, not_last)

            @pl.when(not_last)
            def _():
                stage(2 * gp + 3, src_b, dst_b, sem_ib)

        plsc.subcore_barrier()
        pltpu.sync_copy(acc.at[slab], agg_out.at[cid, slab])
        if with_counts:
            pltpu.sync_copy(hist, cnt_out.at[cid, sid])

    return pl.kernel(body, out_type=out_type, mesh=mesh,
                     scratch_types=scratch,
                     compiler_params=pltpu.CompilerParams(
                         needs_layout_passes=False))


def _make_label_gather_kernel():
    mesh = plsc.VectorSubcoreMesh(core_axis_name="c", subcore_axis_name="s")
    out_type = [jax.ShapeDtypeStruct((L, 2 * D), jnp.float32)]
    scratch = [
        pltpu.VMEM((L_PER_TILE,), jnp.int32),
        pltpu.VMEM((CHUNK, D), jnp.float32),
        pltpu.VMEM((CHUNK, D), jnp.float32),
        pltpu.SemaphoreType.DMA,
        pltpu.SemaphoreType.DMA,
    ]
    tail = L % CHUNK

    def body(table, idx_all, out, idx_v, buf_a, buf_b, sem_a, sem_b):
        cid = lax.axis_index("c")
        sid = lax.axis_index("s")
        base = pl.multiple_of(sid * L_PER_TILE, CHUNK)
        cols = pl.ds(pl.multiple_of(cid * D, D), D)
        pltpu.sync_copy(idx_all.at[cid, pl.ds(base, L_PER_TILE)], idx_v)

        def start(j, buf, sem):
            pltpu.async_copy(table.at[idx_v.at[pl.ds(j * CHUNK, CHUNK)]],
                             buf, sem)

        def wait(buf, sem):
            pltpu.make_async_copy(table.at[idx_v.at[pl.ds(0, CHUNK)]],
                                  buf, sem).wait()

        def put(j, buf):
            row0 = pl.multiple_of(base + j * CHUNK, CHUNK)

            @pl.when(row0 + CHUNK <= L)
            def _():
                pltpu.sync_copy(buf, out.at[pl.ds(row0, CHUNK), cols])

            @pl.when(jnp.logical_and(row0 < L, row0 + CHUNK > L))
            def _():
                pltpu.sync_copy(buf.at[pl.ds(0, tail)],
                                out.at[pl.ds(row0, tail), cols])

        start(0, buf_a, sem_a)
        start(1, buf_b, sem_b)

        @pl.loop(0, (L_CH_TILE - 1) // 2)
        def _(g):
            j0 = g * 2
            j1 = j0 + 1
            wait(buf_a, sem_a)
            put(j0, buf_a)

            @pl.when(j0 + 2 < L_CH_TILE)
            def _():
                start(j0 + 2, buf_a, sem_a)

            wait(buf_b, sem_b)
            put(j1, buf_b)

            @pl.when(j1 + 2 < L_CH_TILE)
            def _():
                start(j1 + 2, buf_b, sem_b)

        wait(buf_a, sem_a)
        put(L_CH_TILE - 1, buf_a)

    return pl.kernel(body, out_type=out_type, mesh=mesh,
                     scratch_types=scratch)


_agg_with_counts = _make_agg_kernel(True)
_agg_plain = _make_agg_kernel(False)
_label_gather = _make_label_gather_kernel()


def _sage_update_body(relu, agg, cnt, x, wl, wr, b, o):
    count = jnp.maximum(jnp.sum(cnt[0], axis=0), 1.0)[:, None]
    mean = agg[0] / count
    acc = (jnp.dot(mean, wl[0], preferred_element_type=jnp.float32) + b[0]
           + jnp.dot(x[0], wr[0], preferred_element_type=jnp.float32))
    o[0] = jnp.maximum(acc, 0.0) if relu else acc


def _sage_update(agg, cnt, x, wl, wr, b, relu):
    return pl.pallas_call(
        functools.partial(_sage_update_body, relu),
        grid=(2,),
        in_specs=[
            pl.BlockSpec((1, ACC_ROWS, D), lambda t: (t, 0, 0)),
            pl.BlockSpec((1, NS, ACC_ROWS), lambda t: (t, 0, 0)),
            pl.BlockSpec((1, ACC_ROWS, D), lambda t: (t, 0, 0)),
            pl.BlockSpec((1, D, D), lambda t: (t, 0, 0)),
            pl.BlockSpec((1, D, D), lambda t: (t, 0, 0)),
            pl.BlockSpec((1, 1, D), lambda t: (t, 0, 0)),
        ],
        out_specs=pl.BlockSpec((1, ACC_ROWS, D), lambda t: (t, 0, 0)),
        out_shape=jax.ShapeDtypeStruct((2, ACC_ROWS, D), jnp.float32),
    )(agg, cnt, x, wl, wr, b)


def _decoder_body(z1, w1, b1, w2, b2, z2o, z3o):
    z2 = jnp.maximum(
        jnp.dot(z1[...], w1[...], preferred_element_type=jnp.float32)
        + b1[...], 0.0)
    z2o[...] = z2
    z3o[...] = jnp.sum(z2 * w2[...], axis=1, keepdims=True) + b2[...]


def _decoder(z1, w1, b1, w2row, b2, rows_block=2000):
    nb = L // rows_block
    return pl.pallas_call(
        _decoder_body,
        grid=(nb,),
        in_specs=[
            pl.BlockSpec((rows_block, 2 * D), lambda i: (i, 0)),
            pl.BlockSpec((2 * D, D), lambda i: (0, 0)),
            pl.BlockSpec((1, D), lambda i: (0, 0)),
            pl.BlockSpec((1, D), lambda i: (0, 0)),
            pl.BlockSpec((1, 1), lambda i: (0, 0)),
        ],
        out_specs=[
            pl.BlockSpec((rows_block, D), lambda i: (i, 0)),
            pl.BlockSpec((rows_block, 1), lambda i: (i, 0)),
        ],
        out_shape=[
            jax.ShapeDtypeStruct((L, D), jnp.float32),
            jax.ShapeDtypeStruct((L, 1), jnp.float32),
        ],
    )(z1, w1, b1, w2row, b2)


def _pad_idx(idx, pad_value, total):
    pad = total - idx.shape[0]
    return jnp.concatenate(
        [idx.astype(jnp.int32),
         jnp.full((pad,), pad_value, dtype=jnp.int32)])


def kernel(x_author, x_hotel, edge_index_author_hotel,
           edge_index_hotel_author, edge_label_index,
           Wl_ah0, Wr_ah0, b_ah0, Wl_ha0, Wr_ha0, b_ha0,
           Wl_ah1, Wr_ah1, b_ah1, Wl_ha1, Wr_ha1, b_ha1,
           dec_W1, dec_b1, dec_W2, dec_b2):
    x_stack = jnp.concatenate(
        [jnp.stack([x_author, x_hotel]),
         jnp.zeros((2, ACC_ROWS - N, D), jnp.float32)], axis=1)
    table0 = x_stack.reshape(2 * ACC_ROWS, D)

    src0 = _pad_idx(edge_index_hotel_author[0] + ACC_ROWS, 0, E_PAD)
    dst0 = _pad_idx(edge_index_hotel_author[1], DUMMY, E_PAD)
    src1 = _pad_idx(edge_index_author_hotel[0], 0, E_PAD)
    dst1 = _pad_idx(edge_index_author_hotel[1], DUMMY, E_PAD)
    src_all = jnp.stack([src0, src1]).reshape(NC, E_PAD // CHUNK, CHUNK)
    dst_all = jnp.stack([dst0, dst1]).reshape(NC, E_PAD // CHUNK, CHUNK)

    zf = jnp.zeros((ACC_ROWS, D), jnp.float32)

    agg0, cnt = _agg_with_counts(table0, src_all, dst_all, zf)
    wl0 = jnp.stack([Wl_ha0, Wl_ah0])
    wr0 = jnp.stack([Wr_ha0, Wr_ah0])
    bb0 = jnp.stack([b_ha0, b_ah0]).reshape(2, 1, D)
    h_stack = _sage_update(agg0, cnt, x_stack, wl0, wr0, bb0, relu=True)

    (agg1,) = _agg_plain(h_stack.reshape(2 * ACC_ROWS, D), src_all,
                         dst_all, zf)
    wl1 = jnp.stack([Wl_ha1, Wl_ah1])
    wr1 = jnp.stack([Wr_ha1, Wr_ah1])
    bb1 = jnp.stack([b_ha1, b_ah1]).reshape(2, 1, D)
    z_stack = _sage_update(agg1, cnt, h_stack, wl1, wr1, bb1, relu=False)

    row = _pad_idx(edge_label_index[0], 0, L_PAD)
    col = _pad_idx(edge_label_index[1] + ACC_ROWS, 0, L_PAD)
    idx_all = jnp.stack([row, col])
    (z1,) = _label_gather(z_stack.reshape(2 * ACC_ROWS, D), idx_all)

    z2, z3 = _decoder(z1, dec_W1, dec_b1.reshape(1, D),
                      dec_W2.reshape(1, D), dec_b2.reshape(1, 1))
    return (z3.reshape(-1), z1, z2)
